# baseline (speedup 1.0000x reference)
import functools

import jax
import jax.numpy as jnp
from jax import lax
from jax.experimental import pallas as pl
from jax.experimental.pallas import tpu as pltpu
from jax.experimental.pallas import tpu_sc as plsc

N = 10000
E = 320000
D = 128

NC = 2
NS = 16
NW = NC * NS

CH = 128
HC = CH // 2
NCHUNK = 80
NST = 2
SB = NCHUNK // NST
EPW = CH * NCHUNK
EPAD = EPW * NW
NPAD = 10240
STRIPE = NPAD // NS

_mesh = plsc.VectorSubcoreMesh(core_axis_name="c", subcore_axis_name="s")


@functools.partial(
    pl.kernel,
    out_type=jax.ShapeDtypeStruct((NC, NPAD, D), jnp.float32),
    mesh=_mesh,
    scratch_types=[
        pltpu.VMEM((2 * SB, CH), jnp.int32),
        pltpu.VMEM((CH, D), jnp.float32),
        pltpu.VMEM((CH, D), jnp.float32),
        pltpu.VMEM_SHARED((NPAD, D), jnp.float32),
        pltpu.SemaphoreType.DMA,
        pltpu.SemaphoreType.DMA,
        pltpu.SemaphoreType.DMA,
        pltpu.SemaphoreType.DMA,
    ],
)
def _sc_agg(h_hbm, idx_hbm, zeros_hbm, out_hbm,
            idx_v, rows0_v, rows1_v, agg_sh, gs00, gs01, gs10, gs11):
    c = lax.axis_index("c")
    s = lax.axis_index("s")
    wid = c * NS + s
    stripe = pl.ds(s * STRIPE, STRIPE)

    def gather_start(j, hf, buf, sem):
        pltpu.async_copy(
            h_hbm.at[idx_v.at[j, pl.ds(hf * HC, HC)]],
            buf.at[pl.ds(hf * HC, HC)], sem)

    def gather_wait(j, hf, buf, sem):
        pltpu.make_async_copy(
            h_hbm.at[idx_v.at[j, pl.ds(hf * HC, HC)]],
            buf.at[pl.ds(hf * HC, HC)], sem).wait()

    def gathers_start(j, buf, s0, s1):
        gather_start(j, 0, buf, s0)
        gather_start(j, 1, buf, s1)

    def gathers_wait(j, buf, s0, s1):
        gather_wait(j, 0, buf, s0)
        gather_wait(j, 1, buf, s1)

    def scatter(j, buf):
        pltpu.sync_copy(buf, agg_sh.at[idx_v.at[SB + j]], add=True)

    row0 = wid * NST * 2 * SB
    for st in range(NST):
        pltpu.sync_copy(idx_hbm.at[pl.ds(row0 + st * 2 * SB, 2 * SB)], idx_v)

        gathers_start(0, rows0_v, gs00, gs01)
        gathers_start(1, rows1_v, gs10, gs11)

        if st == 0:
            @pl.when(c == 0)
            def _():
                pltpu.sync_copy(h_hbm.at[stripe], agg_sh.at[stripe])

            @pl.when(c != 0)
            def _():
                pltpu.sync_copy(zeros_hbm, agg_sh.at[stripe])

            plsc.subcore_barrier()

        def group(k, carry):
            j0 = 2 * k
            j1 = j0 + 1
            gathers_wait(j0, rows0_v, gs00, gs01)
            scatter(j0, rows0_v)
            gathers_start(j0 + 2, rows0_v, gs00, gs01)
            gathers_wait(j1, rows1_v, gs10, gs11)
            scatter(j1, rows1_v)
            gathers_start(j1 + 2, rows1_v, gs10, gs11)
            return carry

        lax.fori_loop(0, SB // 2 - 1, group, 0)

        jt = SB - 2
        gathers_wait(jt, rows0_v, gs00, gs01)
        scatter(jt, rows0_v)
        gathers_wait(jt + 1, rows1_v, gs10, gs11)
        scatter(jt + 1, rows1_v)

    plsc.subcore_barrier()
    pltpu.sync_copy(agg_sh.at[stripe], out_hbm.at[c, stripe])


def _mlp_call(parts, w1, b1, w2, b2, relu_out, out_rows, blk):
    grid = out_rows // blk

    def body(p_ref, w1_ref, b1_ref, w2_ref, b2_ref, o_ref):
        z = p_ref[0] + p_ref[1]
        z1 = jnp.maximum(
            jnp.dot(z, w1_ref[...], preferred_element_type=jnp.float32)
            + b1_ref[...], 0.0)
        z2 = (jnp.dot(z1, w2_ref[...], preferred_element_type=jnp.float32)
              + b2_ref[...])
        o_ref[...] = jnp.maximum(z2, 0.0) if relu_out else z2

    return pl.pallas_call(
        body,
        grid=(grid,),
        in_specs=[
            pl.BlockSpec((NC, blk, D), lambda i: (0, i, 0)),
            pl.BlockSpec((D, D), lambda i: (0, 0)),
            pl.BlockSpec((1, D), lambda i: (0, 0)),
            pl.BlockSpec((D, D), lambda i: (0, 0)),
            pl.BlockSpec((1, D), lambda i: (0, 0)),
        ],
        out_specs=pl.BlockSpec((blk, D), lambda i: (i, 0)),
        out_shape=jax.ShapeDtypeStruct((out_rows, D), jnp.float32),
    )(parts, w1, b1.reshape(1, D), w2, b2.reshape(1, D))


def kernel(features, edge_index,
           W1_0, b1_0, W2_0, b2_0,
           W1_1, b1_1, W2_1, b2_1,
           W1_2, b1_2, W2_2, b2_2):
    src = edge_index[0]
    dst = edge_index[1]
    pad = N + (jnp.arange(EPAD - E, dtype=jnp.int32) % (NPAD - N))
    srcp = jnp.concatenate([src, pad]).reshape(NW, NST, SB, CH)
    dstp = jnp.concatenate([dst, pad]).reshape(NW, NST, SB, CH)
    idxp = jnp.stack([srcp, dstp], axis=2).reshape(NW * NST * 2 * SB, CH)
    zeros = jnp.zeros((STRIPE, D), dtype=jnp.float32)

    h = jnp.concatenate(
        [features, jnp.zeros((NPAD - N, D), dtype=jnp.float32)], axis=0)

    params = [(W1_0, b1_0, W2_0, b2_0),
              (W1_1, b1_1, W2_1, b2_1),
              (W1_2, b1_2, W2_2, b2_2)]
    for i, (w1, b1, w2, b2) in enumerate(params):
        parts = _sc_agg(h, idxp, zeros)
        last = i == len(params) - 1
        h = _mlp_call(parts, w1, b1, w2, b2,
                      relu_out=not last,
                      out_rows=N if last else NPAD,
                      blk=2000 if last else 2048)
    return h


# speedup vs baseline: 1.0179x; 1.0179x over previous
import functools

import jax
import jax.numpy as jnp
from jax import lax
from jax.experimental import pallas as pl
from jax.experimental.pallas import tpu as pltpu
from jax.experimental.pallas import tpu_sc as plsc

N = 10000
E = 320000
D = 128

NC = 2
NS = 16
NW = NC * NS

CH = 128
NCHUNK = 80
NST = 2
SB = NCHUNK // NST
EPW = CH * NCHUNK
EPAD = EPW * NW
NPAD = 10240
STRIPE = NPAD // NS

_mesh = plsc.VectorSubcoreMesh(core_axis_name="c", subcore_axis_name="s")


@functools.partial(
    pl.kernel,
    out_type=jax.ShapeDtypeStruct((NC, NPAD, D), jnp.float32),
    mesh=_mesh,
    scratch_types=[
        pltpu.VMEM((2 * SB, CH), jnp.int32),
        pltpu.VMEM((CH, D), jnp.float32),
        pltpu.VMEM((CH, D), jnp.float32),
        pltpu.VMEM_SHARED((NPAD, D), jnp.float32),
        pltpu.SemaphoreType.DMA,
        pltpu.SemaphoreType.DMA,
    ],
)
def _sc_agg(h_hbm, idx_hbm, zeros_hbm, out_hbm,
            idx_v, rows0_v, rows1_v, agg_sh, sem0, sem1):
    c = lax.axis_index("c")
    s = lax.axis_index("s")
    wid = c * NS + s
    stripe = pl.ds(s * STRIPE, STRIPE)

    def gather_start(j, buf, sem):
        pltpu.async_copy(h_hbm.at[idx_v.at[j]], buf, sem)

    def gather_wait(j, buf, sem):
        pltpu.make_async_copy(h_hbm.at[idx_v.at[j]], buf, sem).wait()

    def scatter(j, buf):
        pltpu.sync_copy(buf, agg_sh.at[idx_v.at[SB + j]], add=True)

    row0 = wid * NST * 2 * SB
    for st in range(NST):
        pltpu.sync_copy(idx_hbm.at[pl.ds(row0 + st * 2 * SB, 2 * SB)], idx_v)

        gather_start(0, rows0_v, sem0)
        gather_start(1, rows1_v, sem1)

        if st == 0:
            @pl.when(c == 0)
            def _():
                pltpu.sync_copy(h_hbm.at[stripe], agg_sh.at[stripe])

            @pl.when(c != 0)
            def _():
                pltpu.sync_copy(zeros_hbm, agg_sh.at[stripe])

            plsc.subcore_barrier()

        def group(k, carry):
            j0 = 2 * k
            j1 = j0 + 1
            gather_wait(j0, rows0_v, sem0)
            scatter(j0, rows0_v)
            gather_start(j0 + 2, rows0_v, sem0)
            gather_wait(j1, rows1_v, sem1)
            scatter(j1, rows1_v)
            gather_start(j1 + 2, rows1_v, sem1)
            return carry

        lax.fori_loop(0, SB // 2 - 1, group, 0)

        jt = SB - 2
        gather_wait(jt, rows0_v, sem0)
        scatter(jt, rows0_v)
        gather_wait(jt + 1, rows1_v, sem1)
        scatter(jt + 1, rows1_v)

    plsc.subcore_barrier()
    pltpu.sync_copy(agg_sh.at[stripe], out_hbm.at[c, stripe])


def _mlp_call(parts, w1, b1, w2, b2, relu_out, out_rows, blk):
    grid = out_rows // blk

    def body(p_ref, w1_ref, b1_ref, w2_ref, b2_ref, o_ref):
        z = p_ref[0] + p_ref[1]
        z1 = jnp.maximum(
            jnp.dot(z, w1_ref[...], preferred_element_type=jnp.float32)
            + b1_ref[...], 0.0)
        z2 = (jnp.dot(z1, w2_ref[...], preferred_element_type=jnp.float32)
              + b2_ref[...])
        o_ref[...] = jnp.maximum(z2, 0.0) if relu_out else z2

    return pl.pallas_call(
        body,
        grid=(grid,),
        in_specs=[
            pl.BlockSpec((NC, blk, D), lambda i: (0, i, 0)),
            pl.BlockSpec((D, D), lambda i: (0, 0)),
            pl.BlockSpec((1, D), lambda i: (0, 0)),
            pl.BlockSpec((D, D), lambda i: (0, 0)),
            pl.BlockSpec((1, D), lambda i: (0, 0)),
        ],
        out_specs=pl.BlockSpec((blk, D), lambda i: (i, 0)),
        out_shape=jax.ShapeDtypeStruct((out_rows, D), jnp.float32),
    )(parts, w1, b1.reshape(1, D), w2, b2.reshape(1, D))


def kernel(features, edge_index,
           W1_0, b1_0, W2_0, b2_0,
           W1_1, b1_1, W2_1, b2_1,
           W1_2, b1_2, W2_2, b2_2):
    src = edge_index[0]
    dst = edge_index[1]
    pad = N + (jnp.arange(EPAD - E, dtype=jnp.int32) % (NPAD - N))
    srcp = jnp.concatenate([src, pad]).reshape(NW, NST, SB, CH)
    dstp = jnp.concatenate([dst, pad]).reshape(NW, NST, SB, CH)
    idxp = jnp.stack([srcp, dstp], axis=2).reshape(NW * NST * 2 * SB, CH)
    zeros = jnp.zeros((STRIPE, D), dtype=jnp.float32)

    h = jnp.concatenate(
        [features, jnp.zeros((NPAD - N, D), dtype=jnp.float32)], axis=0)

    params = [(W1_0, b1_0, W2_0, b2_0),
              (W1_1, b1_1, W2_1, b2_1),
              (W1_2, b1_2, W2_2, b2_2)]
    for i, (w1, b1, w2, b2) in enumerate(params):
        parts = _sc_agg(h, idxp, zeros)
        last = i == len(params) - 1
        h = _mlp_call(parts, w1, b1, w2, b2,
                      relu_out=not last,
                      out_rows=N if last else NPAD,
                      blk=2000 if last else 2048)
    return h
